# baseline (speedup 1.0000x reference)
import functools

import jax
import jax.numpy as jnp
from jax import lax
from jax.experimental import pallas as pl
from jax.experimental.pallas import tpu as pltpu
from jax.experimental.pallas import tpu_sc as plsc

WORD_VOCAB = 1000000
POS_VOCAB = 50
WORD_DIM = 100
POS_DIM = 25
HIDDEN = 200
OUT = 75
B = 16384
T = 6
BT = B * T

NC, NS = 2, 16
NW = NC * NS
PER_W = BT // NW
GROUP = 128
CHUNK = 768
G_PER_CHUNK = CHUNK // GROUP
N_CHUNKS = PER_W // CHUNK
G_PER_W = PER_W // GROUP


def _sc_gather(word_table, pos_table, widx, pidx):
    mesh = plsc.VectorSubcoreMesh(core_axis_name="c", subcore_axis_name="s")

    @functools.partial(
        pl.kernel,
        out_type=[
            jax.ShapeDtypeStruct((BT, WORD_DIM), jnp.float32),
            jax.ShapeDtypeStruct((BT, POS_DIM), jnp.float32),
        ],
        mesh=mesh,
        compiler_params=pltpu.CompilerParams(use_tc_tiling_on_sc=False),
        scratch_types=[
            pltpu.VMEM((G_PER_W, GROUP), jnp.int32),
            pltpu.VMEM((G_PER_W, GROUP), jnp.int32),
            pltpu.VMEM((CHUNK, WORD_DIM), jnp.float32),
            pltpu.VMEM((CHUNK, POS_DIM), jnp.float32),
            pltpu.SemaphoreType.DMA,
        ],
    )
    def gather_k(wt_hbm, pt_hbm, widx_hbm, pidx_hbm, wout_hbm, pout_hbm,
                 widx_v, pidx_v, wrows_v, prows_v, sem):
        wid = lax.axis_index("s") * NC + lax.axis_index("c")
        pltpu.sync_copy(widx_hbm.at[pl.ds(wid * G_PER_W, G_PER_W)], widx_v)
        pltpu.sync_copy(pidx_hbm.at[pl.ds(wid * G_PER_W, G_PER_W)], pidx_v)

        def chunk_body(c, carry):
            copies = []
            for j in range(G_PER_CHUNK):
                g = c * G_PER_CHUNK + j
                copies.append(pltpu.async_copy(
                    wt_hbm.at[widx_v.at[g]],
                    wrows_v.at[pl.ds(j * GROUP, GROUP)], sem))
                copies.append(pltpu.async_copy(
                    pt_hbm.at[pidx_v.at[g]],
                    prows_v.at[pl.ds(j * GROUP, GROUP)], sem))
            for cp in copies:
                cp.wait()
            off = wid * PER_W + c * CHUNK
            pltpu.sync_copy(wrows_v, wout_hbm.at[pl.ds(off, CHUNK)])
            pltpu.sync_copy(prows_v, pout_hbm.at[pl.ds(off, CHUNK)])
            return carry

        lax.fori_loop(0, N_CHUNKS, chunk_body, 0, unroll=False)

    return gather_k(word_table, pos_table, widx, pidx)


def _mlp_body(xw_ref, xp_ref, a_ref, c_ref, b1_ref, w2t_ref, b2_ref, out_ref):
    h = jnp.dot(xw_ref[...], a_ref[...], preferred_element_type=jnp.float32)
    h = h + jnp.dot(xp_ref[...], c_ref[...], preferred_element_type=jnp.float32)
    h = jnp.maximum(h + b1_ref[...], 0.0)
    o = jnp.dot(h, w2t_ref[...], preferred_element_type=jnp.float32) + b2_ref[...]
    m = jnp.max(o, axis=1, keepdims=True)
    e = jnp.exp(o - m)
    lse = jnp.log(jnp.sum(e, axis=1, keepdims=True))
    out_ref[...] = (o - m) - lse


def _tc_mlp(xw, xp, a, c, b1, w2t, b2):
    bm = 2048
    grid = (B // bm,)
    return pl.pallas_call(
        _mlp_body,
        grid=grid,
        in_specs=[
            pl.BlockSpec((bm, T * WORD_DIM), lambda i: (i, 0)),
            pl.BlockSpec((bm, T * POS_DIM), lambda i: (i, 0)),
            pl.BlockSpec((T * WORD_DIM, HIDDEN), lambda i: (0, 0)),
            pl.BlockSpec((T * POS_DIM, HIDDEN), lambda i: (0, 0)),
            pl.BlockSpec((1, HIDDEN), lambda i: (0, 0)),
            pl.BlockSpec((HIDDEN, OUT), lambda i: (0, 0)),
            pl.BlockSpec((1, OUT), lambda i: (0, 0)),
        ],
        out_specs=pl.BlockSpec((bm, OUT), lambda i: (i, 0)),
        out_shape=jax.ShapeDtypeStruct((B, OUT), jnp.float32),
    )(xw, xp, a, c, b1, w2t, b2)


def kernel(word_ids, pos_ids, word_table, pos_table, W1, b1, W2, b2):
    widx = word_ids.astype(jnp.int32).reshape(BT // GROUP, GROUP)
    pidx = pos_ids.astype(jnp.int32).reshape(BT // GROUP, GROUP)

    wrows, prows = _sc_gather(word_table, pos_table, widx, pidx)
    xw = wrows.reshape(B, T * WORD_DIM)
    xp = prows.reshape(B, T * POS_DIM)

    w1t = W1.T.reshape(T, WORD_DIM + POS_DIM, HIDDEN)
    a = w1t[:, :WORD_DIM, :].reshape(T * WORD_DIM, HIDDEN)
    c = w1t[:, WORD_DIM:, :].reshape(T * POS_DIM, HIDDEN)

    return _tc_mlp(xw, xp, a, c, b1.reshape(1, HIDDEN), W2.T, b2.reshape(1, OUT))


# speedup vs baseline: 4.8363x; 4.8363x over previous
import functools

import jax
import jax.numpy as jnp
from jax import lax
from jax.experimental import pallas as pl
from jax.experimental.pallas import tpu as pltpu
from jax.experimental.pallas import tpu_sc as plsc

WORD_VOCAB = 1000000
POS_VOCAB = 50
WORD_DIM = 100
POS_DIM = 25
HIDDEN = 200
OUT = 75
B = 16384
T = 6
BT = B * T
WPAD = 128

NC, NS = 2, 16
NW = NC * NS
PER_W = BT // NW
CHUNK = 256
N_CHUNKS = PER_W // CHUNK


def _sc_gather(word_table, widx):
    mesh = plsc.VectorSubcoreMesh(core_axis_name="c", subcore_axis_name="s")

    @functools.partial(
        pl.kernel,
        out_type=jax.ShapeDtypeStruct((BT, WPAD), jnp.float32),
        mesh=mesh,
        scratch_types=[
            pltpu.VMEM((CHUNK,), jnp.int32),
            pltpu.VMEM((CHUNK, WPAD), jnp.float32),
            pltpu.SemaphoreType.DMA,
        ],
    )
    def gather_k(wt_hbm, widx_hbm, wout_hbm, idx_v, wbuf_v, sem):
        wid = lax.axis_index("s") * NC + lax.axis_index("c")
        base = wid * PER_W

        zeros16 = jnp.zeros((16,), jnp.float32)

        def zrow(r, carry):
            wbuf_v[r, pl.ds(96, 16)] = zeros16
            wbuf_v[r, pl.ds(112, 16)] = zeros16
            return carry

        lax.fori_loop(0, CHUNK, zrow, 0, unroll=False)

        def chunk_body(c, carry):
            off = base + c * CHUNK
            pltpu.sync_copy(widx_hbm.at[pl.ds(off, CHUNK)], idx_v)

            copies = []
            for g in range(CHUNK // 16):
                v = idx_v[pl.ds(g * 16, 16)]
                for j in range(16):
                    copies.append(pltpu.async_copy(
                        wt_hbm.at[v[j]],
                        wbuf_v.at[g * 16 + j, pl.ds(0, WORD_DIM)], sem))
            for cp in copies:
                cp.wait()
            pltpu.sync_copy(wbuf_v, wout_hbm.at[pl.ds(off, CHUNK)])
            return carry

        lax.fori_loop(0, N_CHUNKS, chunk_body, 0, unroll=False)

    return gather_k(word_table, widx)


def _mlp_body(xw_ref, pid_ref, a_ref, cp_ref, pt_ref, b1_ref, w2t_ref, b2_ref,
              out_ref):
    h = jnp.dot(xw_ref[...], a_ref[...], preferred_element_type=jnp.float32)
    iota50 = lax.broadcasted_iota(jnp.int32, (1, POS_VOCAB), 1)
    for t in range(T):
        p_t = jnp.dot(pt_ref[...], cp_ref[t],
                      preferred_element_type=jnp.float32)
        oh_t = (pid_ref[:, t:t + 1] == iota50).astype(jnp.float32)
        h = h + jnp.dot(oh_t, p_t, preferred_element_type=jnp.float32)
    h = jnp.maximum(h + b1_ref[...], 0.0)
    o = jnp.dot(h, w2t_ref[...], preferred_element_type=jnp.float32) + b2_ref[...]
    m = jnp.max(o, axis=1, keepdims=True)
    e = jnp.exp(o - m)
    lse = jnp.log(jnp.sum(e, axis=1, keepdims=True))
    out_ref[...] = (o - m) - lse


def _tc_mlp(xw, pos_ids, a, cp, pt, b1, w2t, b2):
    bm = 2048
    grid = (B // bm,)
    return pl.pallas_call(
        _mlp_body,
        grid=grid,
        in_specs=[
            pl.BlockSpec((bm, T * WPAD), lambda i: (i, 0)),
            pl.BlockSpec((bm, T), lambda i: (i, 0)),
            pl.BlockSpec((T * WPAD, HIDDEN), lambda i: (0, 0)),
            pl.BlockSpec((T, POS_DIM, HIDDEN), lambda i: (0, 0, 0)),
            pl.BlockSpec((POS_VOCAB, POS_DIM), lambda i: (0, 0)),
            pl.BlockSpec((1, HIDDEN), lambda i: (0, 0)),
            pl.BlockSpec((HIDDEN, OUT), lambda i: (0, 0)),
            pl.BlockSpec((1, OUT), lambda i: (0, 0)),
        ],
        out_specs=pl.BlockSpec((bm, OUT), lambda i: (i, 0)),
        out_shape=jax.ShapeDtypeStruct((B, OUT), jnp.float32),
    )(xw, pos_ids, a, cp, pt, b1, w2t, b2)


def kernel(word_ids, pos_ids, word_table, pos_table, W1, b1, W2, b2):
    widx = word_ids.astype(jnp.int32).reshape(BT)

    wrows = _sc_gather(word_table, widx)
    xw = wrows.reshape(B, T * WPAD)

    w1t = W1.T.reshape(T, WORD_DIM + POS_DIM, HIDDEN)
    a = jnp.pad(w1t[:, :WORD_DIM, :],
                ((0, 0), (0, WPAD - WORD_DIM), (0, 0))).reshape(
                    T * WPAD, HIDDEN)
    cp = w1t[:, WORD_DIM:, :]

    return _tc_mlp(xw, pos_ids.astype(jnp.int32), a, cp, pos_table,
                   b1.reshape(1, HIDDEN), W2.T, b2.reshape(1, OUT))


# speedup vs baseline: 5.2857x; 1.0929x over previous
import functools

import jax
import jax.numpy as jnp
from jax import lax
from jax.experimental import pallas as pl
from jax.experimental.pallas import tpu as pltpu
from jax.experimental.pallas import tpu_sc as plsc

WORD_VOCAB = 1000000
POS_VOCAB = 50
WORD_DIM = 100
POS_DIM = 25
HIDDEN = 200
OUT = 75
B = 16384
T = 6
BT = B * T
WPAD = 128

NC, NS = 2, 16
NW = NC * NS
B_PER_W = B // NW
BCHUNK = 64
LCHUNK = BCHUNK * T
N_CHUNKS = B_PER_W // BCHUNK


def _sc_gather(word_table, widx):
    mesh = plsc.VectorSubcoreMesh(core_axis_name="c", subcore_axis_name="s")

    @functools.partial(
        pl.kernel,
        out_type=jax.ShapeDtypeStruct((B, T * WPAD), jnp.float32),
        mesh=mesh,
        scratch_types=[
            pltpu.VMEM((LCHUNK,), jnp.int32),
            pltpu.VMEM((BCHUNK, T * WPAD), jnp.float32),
            pltpu.SemaphoreType.DMA,
        ],
    )
    def gather_k(wt_hbm, widx_hbm, wout_hbm, idx_v, wbuf_v, sem):
        wid = lax.axis_index("s") * NC + lax.axis_index("c")

        zeros16 = jnp.zeros((16,), jnp.float32)

        def zrow(r, carry):
            for t in range(T):
                wbuf_v[r, pl.ds(t * WPAD + 96, 16)] = zeros16
                wbuf_v[r, pl.ds(t * WPAD + 112, 16)] = zeros16
            return carry

        lax.fori_loop(0, BCHUNK, zrow, 0, unroll=False)

        def chunk_body(c, carry):
            off = wid * B_PER_W * T + c * LCHUNK
            pltpu.sync_copy(widx_hbm.at[pl.ds(off, LCHUNK)], idx_v)

            copies = []
            for g in range(LCHUNK // 16):
                v = idx_v[pl.ds(g * 16, 16)]
                for j in range(16):
                    li = g * 16 + j
                    copies.append(pltpu.async_copy(
                        wt_hbm.at[v[j]],
                        wbuf_v.at[li // T, pl.ds((li % T) * WPAD, WORD_DIM)],
                        sem))
            for cp in copies:
                cp.wait()
            pltpu.sync_copy(
                wbuf_v, wout_hbm.at[pl.ds(wid * B_PER_W + c * BCHUNK, BCHUNK)])
            return carry

        lax.fori_loop(0, N_CHUNKS, chunk_body, 0, unroll=False)

    return gather_k(word_table, widx)


def _mlp_body(xw_ref, pid_ref, a_ref, cp_ref, pt_ref, b1_ref, w2t_ref, b2_ref,
              out_ref):
    h = jnp.dot(xw_ref[...], a_ref[...], preferred_element_type=jnp.float32)
    iota50 = lax.broadcasted_iota(jnp.int32, (1, POS_VOCAB), 1)
    for t in range(T):
        p_t = jnp.dot(pt_ref[...], cp_ref[t],
                      preferred_element_type=jnp.float32)
        oh_t = (pid_ref[:, t:t + 1] == iota50).astype(jnp.float32)
        h = h + jnp.dot(oh_t, p_t, preferred_element_type=jnp.float32)
    h = jnp.maximum(h + b1_ref[...], 0.0)
    o = jnp.dot(h, w2t_ref[...], preferred_element_type=jnp.float32) + b2_ref[...]
    m = jnp.max(o, axis=1, keepdims=True)
    e = jnp.exp(o - m)
    lse = jnp.log(jnp.sum(e, axis=1, keepdims=True))
    out_ref[...] = (o - m) - lse


def _tc_mlp(xw, pos_ids, a, cp, pt, b1, w2t, b2):
    bm = 2048
    grid = (B // bm,)
    return pl.pallas_call(
        _mlp_body,
        grid=grid,
        in_specs=[
            pl.BlockSpec((bm, T * WPAD), lambda i: (i, 0)),
            pl.BlockSpec((bm, T), lambda i: (i, 0)),
            pl.BlockSpec((T * WPAD, HIDDEN), lambda i: (0, 0)),
            pl.BlockSpec((T, POS_DIM, HIDDEN), lambda i: (0, 0, 0)),
            pl.BlockSpec((POS_VOCAB, POS_DIM), lambda i: (0, 0)),
            pl.BlockSpec((1, HIDDEN), lambda i: (0, 0)),
            pl.BlockSpec((HIDDEN, OUT), lambda i: (0, 0)),
            pl.BlockSpec((1, OUT), lambda i: (0, 0)),
        ],
        out_specs=pl.BlockSpec((bm, OUT), lambda i: (i, 0)),
        out_shape=jax.ShapeDtypeStruct((B, OUT), jnp.float32),
    )(xw, pos_ids, a, cp, pt, b1, w2t, b2)


def kernel(word_ids, pos_ids, word_table, pos_table, W1, b1, W2, b2):
    widx = word_ids.astype(jnp.int32).reshape(BT)

    xw = _sc_gather(word_table, widx)

    w1t = W1.T.reshape(T, WORD_DIM + POS_DIM, HIDDEN)
    a = jnp.pad(w1t[:, :WORD_DIM, :],
                ((0, 0), (0, WPAD - WORD_DIM), (0, 0))).reshape(
                    T * WPAD, HIDDEN)
    cp = w1t[:, WORD_DIM:, :]

    return _tc_mlp(xw, pos_ids.astype(jnp.int32), a, cp, pos_table,
                   b1.reshape(1, HIDDEN), W2.T, b2.reshape(1, OUT))
